# baseline (speedup 1.0000x reference)
import functools

import jax
import jax.numpy as jnp
from jax import lax
from jax.experimental import pallas as pl
from jax.experimental.pallas import tpu as pltpu
from jax.experimental.pallas import tpu_sc as plsc

_NB = 4
_NBUF = 2
_GRP = 4
_PAD = 256


@functools.lru_cache(maxsize=None)
def _make_gather(batch: int, hist: int, dim: int, vocab: int):
  info = plsc.get_sparse_core_info()
  lanes = info.num_lanes
  nw = info.num_cores * info.num_subcores
  rows_per_worker = batch // nw
  steps = rows_per_worker // _NB
  groups = steps // _NBUF
  chunk = _NB * hist
  full_vecs = chunk // lanes
  tail = chunk - full_vecs * lanes
  ring = (full_vecs + (1 if tail else 0)) * lanes
  vpl = dim // lanes

  mesh = plsc.VectorSubcoreMesh(core_axis_name="c", subcore_axis_name="s")

  @functools.partial(
      pl.kernel,
      out_type=jax.ShapeDtypeStruct((batch, hist, dim), jnp.float32),
      mesh=mesh,
      scratch_types=[
          pltpu.VMEM((vocab, dim), jnp.float32),
          [pltpu.VMEM((ring,), jnp.int32)] * _NBUF,
          pltpu.VMEM((_NBUF, _NB, hist, dim), jnp.float32),
          [pltpu.SemaphoreType.DMA] * _NBUF,
          [pltpu.SemaphoreType.DMA] * _NBUF,
      ],
  )
  def gather_kernel(table_hbm, idx_hbm, out_hbm, table_v, idx_v, stage,
                    sem_i, sem_o):
    wid = lax.axis_index("s") * info.num_cores + lax.axis_index("c")
    row0 = wid * rows_per_worker
    look0 = row0 * hist

    pltpu.sync_copy(table_hbm, table_v)

    def idx_desc(s, p):
      return pltpu.make_async_copy(
          idx_hbm.at[pl.ds(look0 + s * chunk, ring)], idx_v[p], sem_i[p])

    def write_desc(s, p):
      return pltpu.make_async_copy(
          stage.at[p], out_hbm.at[pl.ds(row0 + s * _NB, _NB)], sem_o[p])

    def copy_rows(p, ls, iv, ks):
      rows = [iv[k] for k in ks]
      vals = [[table_v[row, pl.ds(j * lanes, lanes)] for j in range(vpl)]
              for row in rows]
      for j in range(vpl):
        for i, l in enumerate(ls):
          stage[p, l // hist, l % hist, pl.ds(j * lanes, lanes)] = vals[i][j]

    def build(p):
      for t in range(full_vecs + (1 if tail else 0)):
        iv = idx_v[p][pl.ds(t * lanes, lanes)]
        nk = lanes if t < full_vecs else tail
        for k0 in range(0, nk, _GRP):
          ks = list(range(k0, min(k0 + _GRP, nk)))
          copy_rows(p, [t * lanes + k for k in ks], iv, ks)

    idx_desc(0, 0).start()
    idx_desc(1, 1).start()

    def group(g, carry):
      for p in range(_NBUF):
        s = g * _NBUF + p
        idx_desc(s, p).wait()

        @pl.when(g >= 1)
        def _wait_prev():
          write_desc(s - _NBUF, p).wait()

        build(p)
        write_desc(s, p).start()

        @pl.when(g < groups - 1)
        def _prefetch():
          idx_desc(s + _NBUF, p).start()
      return carry

    lax.fori_loop(0, groups, group, 0)

    for p in range(_NBUF):
      write_desc(steps - _NBUF + p, p).wait()

  return gather_kernel


def kernel(channel_indices, weight):
  batch, hist = channel_indices.shape
  vocab, dim = weight.shape
  idx_flat = jnp.pad(channel_indices.reshape(-1).astype(jnp.int32), (0, _PAD))
  return _make_gather(batch, hist, dim, vocab)(weight, idx_flat)


# speedup vs baseline: 2.1199x; 2.1199x over previous
import functools

import jax
import jax.numpy as jnp
from jax import lax
from jax.experimental import pallas as pl
from jax.experimental.pallas import tpu as pltpu
from jax.experimental.pallas import tpu_sc as plsc

_NB = 4
_NBUF = 2
_PAD = 256


@functools.lru_cache(maxsize=None)
def _make_gather(batch: int, hist: int, dim: int, vocab: int):
  info = plsc.get_sparse_core_info()
  lanes = info.num_lanes
  nw = info.num_cores * info.num_subcores
  rows_per_worker = batch // nw
  steps = rows_per_worker // _NB
  groups = steps // _NBUF
  chunk = _NB * hist
  full_vecs = chunk // lanes
  tail = chunk - full_vecs * lanes
  ring = (full_vecs + (1 if tail else 0)) * lanes
  vpl = dim // lanes

  mesh = plsc.VectorSubcoreMesh(core_axis_name="c", subcore_axis_name="s")

  @functools.partial(
      pl.kernel,
      out_type=jax.ShapeDtypeStruct((batch, hist, dim), jnp.float32),
      mesh=mesh,
      scratch_types=[
          pltpu.VMEM((vocab, dim), jnp.float32),
          [pltpu.VMEM((ring,), jnp.int32)] * _NBUF,
          pltpu.VMEM((_NBUF, _NB, hist, dim), jnp.float32),
          [pltpu.SemaphoreType.DMA] * _NBUF,
          [pltpu.SemaphoreType.DMA] * _NBUF,
      ],
  )
  def gather_kernel(table_hbm, idx_hbm, out_hbm, table_v, idx_v, stage,
                    sem_i, sem_o):
    wid = lax.axis_index("s") * info.num_cores + lax.axis_index("c")
    row0 = wid * rows_per_worker
    look0 = row0 * hist

    pltpu.sync_copy(table_hbm, table_v)

    def idx_desc(s, p):
      return pltpu.make_async_copy(
          idx_hbm.at[pl.ds(look0 + s * chunk, ring)], idx_v[p], sem_i[p])

    def write_desc(s, p):
      return pltpu.make_async_copy(
          stage.at[p], out_hbm.at[pl.ds(row0 + s * _NB, _NB)], sem_o[p])

    def copy_rows(p, pos, iv, ks):
      rows = [iv[k] for k in ks]
      vals = [[table_v[row, pl.ds(j * lanes, lanes)] for j in range(vpl)]
              for row in rows]
      for j in range(vpl):
        for i, (b2, h) in enumerate(pos):
          stage[p, b2, h, pl.ds(j * lanes, lanes)] = vals[i][j]

    def build(p):
      @plsc.parallel_loop(0, full_vecs, 1, unroll=2)
      def t_body(t):
        iv = idx_v[p][pl.ds(t * lanes, lanes)]
        for k0 in range(0, lanes, 4):
          ks = list(range(k0, k0 + 4))
          pos = []
          for k in ks:
            l = t * lanes + k
            b2 = sum((l >= (m + 1) * hist).astype(jnp.int32)
                     for m in range(_NB - 1))
            pos.append((b2, l - b2 * hist))
          copy_rows(p, pos, iv, ks)
      if tail:
        iv = idx_v[p][pl.ds(full_vecs * lanes, lanes)]
        for k0 in range(0, tail, 4):
          ks = list(range(k0, min(k0 + 4, tail)))
          pos = []
          for k in ks:
            l = full_vecs * lanes + k
            pos.append((l // hist, l % hist))
          copy_rows(p, pos, iv, ks)

    idx_desc(0, 0).start()
    idx_desc(1, 1).start()
    for p in range(_NBUF):
      idx_desc(p, p).wait()
      build(p)
      write_desc(p, p).start()
      idx_desc(p + _NBUF, p).start()

    def group(g, carry):
      for p in range(_NBUF):
        s = g * _NBUF + p
        idx_desc(s, p).wait()
        write_desc(s - _NBUF, p).wait()
        build(p)
        write_desc(s, p).start()

        @pl.when(g < groups - 1)
        def _prefetch():
          idx_desc(s + _NBUF, p).start()
      return carry

    lax.fori_loop(1, groups, group, 0)

    for p in range(_NBUF):
      write_desc(steps - _NBUF + p, p).wait()

  return gather_kernel


def kernel(channel_indices, weight):
  batch, hist = channel_indices.shape
  vocab, dim = weight.shape
  idx_flat = jnp.pad(channel_indices.reshape(-1).astype(jnp.int32), (0, _PAD))
  return _make_gather(batch, hist, dim, vocab)(weight, idx_flat)
